# baseline (speedup 1.0000x reference)
import numpy as np

import jax
import jax.numpy as jnp
from jax import lax
from jax.experimental import pallas as pl
from jax.experimental.pallas import tpu as pltpu
from jax.experimental.pallas import tpu_sc as plsc

N = 10000
CIN = 128
COUT = 128
KVOL = 27
BN_EPS = 1e-5
SLOPE = 0.01

NC = 1
NS = 16
GB = 40
NGROUP = 256
NP = NGROUP * GB

FAST_CID = 0
NG_FAST = 16
NG_SLOW = 2
CH_FAST = NG_FAST * GB
CH_SLOW = NG_SLOW * GB


def _assign() -> np.ndarray:
    a = np.zeros((NC * NS, NG_FAST), dtype=np.int32)
    for w in range(NC * NS):
        cid, sid = w % NC, w // NC
        if cid == FAST_CID:
            a[w, :] = np.arange(sid * NG_FAST, (sid + 1) * NG_FAST)
        else:
            g0 = NS * NG_FAST + sid * NG_SLOW
            a[w, :NG_SLOW] = np.arange(g0, g0 + NG_SLOW)
            a[w, NG_SLOW:] = 0
    return a


_BLK = 256


def _mm_body(x_ref, w_ref, y_ref):
    x = x_ref[...]
    for k in range(KVOL):
        y_ref[k] = jnp.dot(x, w_ref[k], preferred_element_type=jnp.float32)


def _stage1(xb, wb):
    return pl.pallas_call(
        _mm_body,
        grid=(NP // _BLK,),
        in_specs=[
            pl.BlockSpec((_BLK, CIN), lambda i: (i, 0)),
            pl.BlockSpec((KVOL, CIN, COUT), lambda i: (0, 0, 0)),
        ],
        out_specs=pl.BlockSpec((KVOL, _BLK, COUT), lambda i: (0, i, 0)),
        out_shape=jax.ShapeDtypeStruct((KVOL, NP, COUT), jnp.float32),
    )(xb, wb)


def _sc_body(y_hbm, idx_hbm, conv_hbm, idx_v, acc, sem):
    cid = lax.axis_index("c")
    sid = lax.axis_index("s")
    w = sid * NC + cid

    pltpu.sync_copy(idx_hbm.at[w], idx_v)

    def fire(k, gsrc, gdst, add):
        return pltpu.async_copy(
            y_hbm.at[idx_v.at[k, gsrc]],
            acc.at[pl.ds(gdst * GB, GB)],
            sem,
            add=add,
        )

    def drain(g):
        pltpu.make_async_copy(
            y_hbm.at[idx_v.at[0, g]],
            acc.at[pl.ds(g * GB, GB)],
            sem,
        ).wait()

    def run(ng, g0, base):
        for g in range(ng):
            fire(0, g0 + g, g, False)

        def body(k, carry):
            for g in range(ng):
                drain(g)
                fire(k, g0 + g, g, True)
            return carry

        lax.fori_loop(1, KVOL, body, 0)
        for g in range(ng):
            drain(g)
        pltpu.sync_copy(
            acc.at[pl.ds(0, ng * GB)], conv_hbm.at[pl.ds(base, ng * GB)]
        )

    for half in range(2):
        run(NG_FAST // 2, half * (NG_FAST // 2),
            sid * CH_FAST + half * (CH_FAST // 2))


def _stage2(y_flat, idxg):
    mesh = plsc.VectorSubcoreMesh(
        core_axis_name="c", subcore_axis_name="s", num_cores=NC,
        num_subcores=NS,
    )
    f = pl.kernel(
        _sc_body,
        out_type=jax.ShapeDtypeStruct((NP, COUT), jnp.float32),
        mesh=mesh,
        scratch_types=[
            pltpu.VMEM((KVOL, NG_FAST, GB), jnp.int32),
            pltpu.VMEM((CH_FAST // 2, COUT), jnp.float32),
            pltpu.SemaphoreType.DMA,
        ],
    )
    return f(y_flat, idxg)


_RB = 400


def _bn_body(c_ref, g_ref, b_ref, o_ref, s_ref, q_ref):
    p = pl.program_id(0)
    i = pl.program_id(1)
    c = c_ref[...]

    @pl.when((p == 0) & (i == 0))
    def _init():
        s_ref[...] = jnp.zeros_like(s_ref)
        q_ref[...] = jnp.zeros_like(q_ref)

    @pl.when(p == 0)
    def _accumulate():
        s_ref[...] += jnp.sum(c, axis=0, keepdims=True)
        q_ref[...] += jnp.sum(c * c, axis=0, keepdims=True)
        o_ref[...] = jnp.zeros_like(o_ref)

    @pl.when(p == 1)
    def _normalize():
        mean = s_ref[...] * (1.0 / N)
        var = q_ref[...] * (1.0 / N) - mean * mean
        inv = lax.rsqrt(var + BN_EPS)
        scale = g_ref[...] * inv
        shift = b_ref[...] - mean * scale
        o = c * scale + shift
        o_ref[...] = jnp.where(o >= 0, o, SLOPE * o)


def _stage3(conv, gamma2, beta2):
    return pl.pallas_call(
        _bn_body,
        grid=(2, N // _RB),
        in_specs=[
            pl.BlockSpec((_RB, COUT), lambda p, i: (i, 0)),
            pl.BlockSpec((1, COUT), lambda p, i: (0, 0)),
            pl.BlockSpec((1, COUT), lambda p, i: (0, 0)),
        ],
        out_specs=pl.BlockSpec((_RB, COUT), lambda p, i: (i, 0)),
        out_shape=jax.ShapeDtypeStruct((N, COUT), jnp.float32),
        scratch_shapes=[
            pltpu.VMEM((1, COUT), jnp.float32),
            pltpu.VMEM((1, COUT), jnp.float32),
        ],
    )(conv, gamma2, beta2)


def kernel(x, neighbor_idx, W, gamma, beta):
    x_pad = jnp.pad(x, ((0, NP - N), (0, 0))).astype(jnp.bfloat16)
    wb = W.astype(jnp.bfloat16)
    y = _stage1(x_pad, wb)
    y_flat = y.reshape(KVOL * NP, COUT)

    idx32 = neighbor_idx.astype(jnp.int32)
    flat = idx32 + (jnp.arange(KVOL, dtype=jnp.int32) * NP)[None, :]
    flat = jnp.pad(flat, ((0, NP - N), (0, 0)))
    groups = flat.reshape(NGROUP, GB, KVOL).transpose(0, 2, 1)
    idxg = groups[jnp.asarray(_assign())]
    idxg = idxg.transpose(0, 2, 1, 3)

    conv = _stage2(y_flat, idxg)
    return _stage3(conv, gamma.reshape(1, -1), beta.reshape(1, -1))


# speedup vs baseline: 1.3664x; 1.3664x over previous
import numpy as np

import jax
import jax.numpy as jnp
from jax import lax
from jax.experimental import pallas as pl
from jax.experimental.pallas import tpu as pltpu
from jax.experimental.pallas import tpu_sc as plsc

N = 10000
CIN = 128
COUT = 128
KVOL = 27
BN_EPS = 1e-5
SLOPE = 0.01

NC = 2
NS = 16
GB = 40
NGROUP = 256
NP = NGROUP * GB

FAST_CID = 0
NG_FAST = 14
NG_SLOW = 2
CH_FAST = NG_FAST * GB
CH_SLOW = NG_SLOW * GB


def _assign() -> np.ndarray:
    a = np.zeros((NC * NS, NG_FAST), dtype=np.int32)
    for w in range(NC * NS):
        cid, sid = w % NC, w // NC
        if cid == FAST_CID:
            a[w, :] = np.arange(sid * NG_FAST, (sid + 1) * NG_FAST)
        else:
            g0 = NS * NG_FAST + sid * NG_SLOW
            a[w, :NG_SLOW] = np.arange(g0, g0 + NG_SLOW)
            a[w, NG_SLOW:] = 0
    return a


_BLK = 256


def _mm_body(x_ref, w_ref, y_ref):
    x = x_ref[...].astype(jnp.bfloat16)
    w = w_ref[...].astype(jnp.bfloat16)
    for k in range(KVOL):
        y_ref[k] = jnp.dot(x, w[k], preferred_element_type=jnp.float32)


def _stage1(xb, wb):
    return pl.pallas_call(
        _mm_body,
        grid=(NP // _BLK,),
        in_specs=[
            pl.BlockSpec((_BLK, CIN), lambda i: (i, 0)),
            pl.BlockSpec((KVOL, CIN, COUT), lambda i: (0, 0, 0)),
        ],
        out_specs=pl.BlockSpec((KVOL, _BLK, COUT), lambda i: (0, i, 0)),
        out_shape=jax.ShapeDtypeStruct((KVOL, NP, COUT), jnp.float32),
    )(xb, wb)


def _sc_body(y_hbm, idx_hbm, conv_hbm, idx_v, acc, sem):
    cid = lax.axis_index("c")
    sid = lax.axis_index("s")
    w = sid * NC + cid

    pltpu.sync_copy(idx_hbm.at[w], idx_v)

    def fire(k, gsrc, gdst, add):
        return pltpu.async_copy(
            y_hbm.at[idx_v.at[k, gsrc]],
            acc.at[pl.ds(gdst * GB, GB)],
            sem,
            add=add,
        )

    def drain(g):
        pltpu.make_async_copy(
            y_hbm.at[idx_v.at[0, g]],
            acc.at[pl.ds(g * GB, GB)],
            sem,
        ).wait()

    def run(ng, g0, base):
        for g in range(ng):
            fire(0, g0 + g, g, False)

        def body(k, carry):
            for g in range(ng):
                drain(g)
                fire(k, g0 + g, g, True)
            return carry

        lax.fori_loop(1, KVOL, body, 0)
        for g in range(ng):
            drain(g)
        pltpu.sync_copy(
            acc.at[pl.ds(0, ng * GB)], conv_hbm.at[pl.ds(base, ng * GB)]
        )

    @pl.when(cid == FAST_CID)
    def _fast():
        run(NG_FAST, 0, sid * CH_FAST)

    @pl.when(cid != FAST_CID)
    def _slow():
        run(NG_SLOW, 0, NS * CH_FAST + sid * CH_SLOW)


def _stage2(y_flat, idxg):
    mesh = plsc.VectorSubcoreMesh(
        core_axis_name="c", subcore_axis_name="s", num_cores=NC,
        num_subcores=NS,
    )
    f = pl.kernel(
        _sc_body,
        out_type=jax.ShapeDtypeStruct((NP, COUT), jnp.float32),
        mesh=mesh,
        scratch_types=[
            pltpu.VMEM((KVOL, NG_FAST, GB), jnp.int32),
            pltpu.VMEM((CH_FAST, COUT), jnp.float32),
            pltpu.SemaphoreType.DMA,
        ],
    )
    return f(y_flat, idxg)


_RB = 2000


def _bn_body(c_ref, g_ref, b_ref, o_ref, s_ref, q_ref):
    p = pl.program_id(0)
    i = pl.program_id(1)
    c = c_ref[...]

    @pl.when((p == 0) & (i == 0))
    def _init():
        s_ref[...] = jnp.zeros_like(s_ref)
        q_ref[...] = jnp.zeros_like(q_ref)

    @pl.when(p == 0)
    def _accumulate():
        s_ref[...] += jnp.sum(c, axis=0, keepdims=True)
        q_ref[...] += jnp.sum(c * c, axis=0, keepdims=True)
        o_ref[...] = jnp.zeros_like(o_ref)

    @pl.when(p == 1)
    def _normalize():
        mean = s_ref[...] * (1.0 / N)
        var = q_ref[...] * (1.0 / N) - mean * mean
        inv = lax.rsqrt(var + BN_EPS)
        scale = g_ref[...] * inv
        shift = b_ref[...] - mean * scale
        o = c * scale + shift
        o_ref[...] = jnp.where(o >= 0, o, SLOPE * o)


def _stage3(conv, gamma2, beta2):
    return pl.pallas_call(
        _bn_body,
        grid=(2, N // _RB),
        in_specs=[
            pl.BlockSpec((_RB, COUT), lambda p, i: (i, 0)),
            pl.BlockSpec((1, COUT), lambda p, i: (0, 0)),
            pl.BlockSpec((1, COUT), lambda p, i: (0, 0)),
        ],
        out_specs=pl.BlockSpec((_RB, COUT), lambda p, i: (i, 0)),
        out_shape=jax.ShapeDtypeStruct((N, COUT), jnp.float32),
        scratch_shapes=[
            pltpu.VMEM((1, COUT), jnp.float32),
            pltpu.VMEM((1, COUT), jnp.float32),
        ],
    )(conv, gamma2, beta2)


def kernel(x, neighbor_idx, W, gamma, beta):
    y = _stage1(x, W)
    y_flat = y.reshape(KVOL * NP, COUT)

    idx32 = neighbor_idx.astype(jnp.int32)
    flat = idx32 + (jnp.arange(KVOL, dtype=jnp.int32) * NP)[None, :]
    flat = jnp.pad(flat, ((0, NP - N), (0, 0)))
    groups = flat.reshape(NGROUP, GB, KVOL).transpose(0, 2, 1)
    idxg = groups[jnp.asarray(_assign())]
    idxg = idxg.transpose(0, 2, 1, 3)

    conv = _stage2(y_flat, idxg)
    return _stage3(conv, gamma.reshape(1, -1), beta.reshape(1, -1))
